# Initial kernel scaffold; baseline (speedup 1.0000x reference)
import jax
import jax.numpy as jnp
from jax.experimental import pallas as pl


def kernel(loc_data, conf_data, bin_conf_data, priors, targets):
    raise NotImplementedError("write your pallas kernel here")


# speedup vs baseline: 31.5622x; 31.5622x over previous
import functools

import jax
import jax.numpy as jnp
from jax.experimental import pallas as pl

_NUM_CLASSES = 81
_THRESHOLD = 0.5
_NEGPOS_RATIO = 3
_VAR0, _VAR1 = 0.1, 0.2
_T = 16
_R, _L = 256, 128
_P = _R * _L


def _topk_mask(mined, k, idx2d):
    u = jax.lax.bitcast_convert_type(mined, jnp.int32)
    v = jnp.int32(0)
    for bit in range(30, -1, -1):
        cand = v | jnp.int32(1 << bit)
        c = jnp.sum((u >= cand).astype(jnp.int32))
        v = jnp.where(c >= k, cand, v)
    c_gt = jnp.sum((u > v).astype(jnp.int32))
    r = k - c_gt
    tied = u == v
    lo = jnp.int32(0)
    for bit in range(14, -1, -1):
        cand = lo | jnp.int32(1 << bit)
        f = jnp.sum((tied & (idx2d < cand)).astype(jnp.int32))
        lo = jnp.where(f < r, cand, lo)
    m = jnp.where(r > 0, lo + 1, jnp.int32(0))
    return ((u > v) | (tied & (idx2d < m))) & (k > 0)


def _sample_kernel(loc_ref, conf_ref, bin_ref, pri_ref, tgt_ref,
                   scal_ref, pos_ref, negb_ref, negm_ref):
    idx2d = (jax.lax.broadcasted_iota(jnp.int32, (_R, _L), 0) * _L
             + jax.lax.broadcasted_iota(jnp.int32, (_R, _L), 1))

    pri = pri_ref[:]
    px1 = pri[0] - pri[2] * 0.5
    py1 = pri[1] - pri[3] * 0.5
    px2 = pri[0] + pri[2] * 0.5
    py2 = pri[1] + pri[3] * 0.5
    area_b = (px2 - px1) * (py2 - py1)

    tgt = tgt_ref[0]

    best_ov = jnp.full((_R, _L), -1.0, jnp.float32)
    best_ti = jnp.zeros((_R, _L), jnp.int32)
    bp_idx = []
    for t in range(_T):
        tx1, ty1, tx2, ty2 = tgt[t, 0], tgt[t, 1], tgt[t, 2], tgt[t, 3]
        iw = jnp.clip(jnp.minimum(tx2, px2) - jnp.maximum(tx1, px1), 0.0, None)
        ih = jnp.clip(jnp.minimum(ty2, py2) - jnp.maximum(ty1, py1), 0.0, None)
        inter = iw * ih
        area_a = (tx2 - tx1) * (ty2 - ty1)
        iou = inter / (area_a + area_b - inter)
        upd = iou > best_ov
        best_ti = jnp.where(upd, t, best_ti)
        best_ov = jnp.where(upd, iou, best_ov)
        mx = jnp.max(iou)
        bp_idx.append(jnp.min(jnp.where(iou == mx, idx2d, _P)))
    for t in range(_T):
        hit = idx2d == bp_idx[t]
        best_ov = jnp.where(hit, 2.0, best_ov)
        best_ti = jnp.where(hit, t, best_ti)

    mx1 = jnp.zeros((_R, _L), jnp.float32)
    my1 = jnp.zeros((_R, _L), jnp.float32)
    mx2 = jnp.zeros((_R, _L), jnp.float32)
    my2 = jnp.zeros((_R, _L), jnp.float32)
    lab = jnp.zeros((_R, _L), jnp.float32)
    for t in range(_T):
        sel = best_ti == t
        mx1 = jnp.where(sel, tgt[t, 0], mx1)
        my1 = jnp.where(sel, tgt[t, 1], my1)
        mx2 = jnp.where(sel, tgt[t, 2], mx2)
        my2 = jnp.where(sel, tgt[t, 3], my2)
        lab = jnp.where(sel, tgt[t, 4], lab)

    pos = best_ov >= _THRESHOLD
    conf_t = jnp.where(pos, lab.astype(jnp.int32) + 1, 0)

    loc = loc_ref[0]
    g0 = ((mx1 + mx2) * 0.5 - pri[0]) / (_VAR0 * pri[2])
    g1 = ((my1 + my2) * 0.5 - pri[1]) / (_VAR0 * pri[3])
    g2 = jnp.log((mx2 - mx1) / pri[2]) / _VAR1
    g3 = jnp.log((my2 - my1) / pri[3]) / _VAR1
    posf = pos.astype(jnp.float32)
    loss_l = jnp.float32(0.0)
    for c, g in enumerate((g0, g1, g2, g3)):
        d = loc[c] - g
        ad = jnp.abs(d)
        loss_l += jnp.sum(jnp.where(ad < 1.0, 0.5 * d * d, ad - 0.5) * posf)

    b = bin_ref[0]
    b0, b1 = b[0], b[1]
    mb = jnp.maximum(b0, b1)
    lse_bin = mb + jnp.log(jnp.exp(b0 - mb) + jnp.exp(b1 - mb))
    ce_bin = lse_bin - jnp.where(pos, b1, b0)

    tgt_cls = conf_t - 1
    sexp = jnp.zeros((_R, _L), jnp.float32)
    cmax = jnp.full((_R, _L), -jnp.inf, jnp.float32)
    conf_sel = jnp.zeros((_R, _L), jnp.float32)
    for c in range(_NUM_CLASSES - 1):
        x = conf_ref[0, c]
        sexp += jnp.exp(x)
        cmax = jnp.maximum(cmax, x)
        conf_sel = jnp.where(tgt_cls == c, x, conf_sel)
    p0 = b0 + jnp.log(sexp)
    m81 = jnp.maximum(p0, cmax + b1)
    b1m = b1 - m81
    acc = jnp.exp(p0 - m81)
    for c in range(_NUM_CLASSES - 1):
        acc += jnp.exp(conf_ref[0, c] + b1m)
    lse_cls = m81 + jnp.log(acc)
    tgt_logit = jnp.where(conf_t == 0, p0, conf_sel + b1)
    ce_cls = lse_cls - tgt_logit

    num_pos = jnp.sum(pos.astype(jnp.int32))
    k = jnp.minimum(_NEGPOS_RATIO * num_pos, _P - 1)

    neg_b = _topk_mask(jnp.where(pos, 0.0, ce_bin), k, idx2d)
    neg_m = _topk_mask(jnp.where(pos, 0.0, ce_cls), k, idx2d)

    loss_bin = jnp.sum(ce_bin * (pos | neg_b).astype(jnp.float32))
    loss_cls = jnp.sum(ce_cls * (pos | neg_m).astype(jnp.float32))

    srow = jax.lax.broadcasted_iota(jnp.int32, (8, 128), 0)
    scol = jax.lax.broadcasted_iota(jnp.int32, (8, 128), 1)
    stile = jnp.where((srow == 0) & (scol == 0), loss_l, 0.0)
    stile = stile + jnp.where((srow == 0) & (scol == 1), loss_bin, 0.0)
    stile = stile + jnp.where((srow == 0) & (scol == 2), loss_cls, 0.0)
    stile = stile + jnp.where((srow == 0) & (scol == 3),
                              num_pos.astype(jnp.float32), 0.0)
    scal_ref[0] = stile
    pos_ref[0] = posf
    negb_ref[0] = neg_b.astype(jnp.float32)
    negm_ref[0] = neg_m.astype(jnp.float32)


@functools.partial(jax.jit, static_argnames=())
def kernel(loc_data, conf_data, bin_conf_data, priors, targets):
    B, P, _ = loc_data.shape
    loc_r = loc_data.transpose(0, 2, 1).reshape(B, 4, _R, _L)
    bin_r = bin_conf_data.transpose(0, 2, 1).reshape(B, 2, _R, _L)
    conf_r = conf_data.transpose(0, 2, 1).reshape(B, _NUM_CLASSES - 1, _R, _L)
    pri_r = priors.transpose(1, 0).reshape(4, _R, _L)

    scal, posf, negbf, negmf = pl.pallas_call(
        _sample_kernel,
        grid=(B,),
        in_specs=[
            pl.BlockSpec((1, 4, _R, _L), lambda b: (b, 0, 0, 0)),
            pl.BlockSpec((1, _NUM_CLASSES - 1, _R, _L), lambda b: (b, 0, 0, 0)),
            pl.BlockSpec((1, 2, _R, _L), lambda b: (b, 0, 0, 0)),
            pl.BlockSpec((4, _R, _L), lambda b: (0, 0, 0)),
            pl.BlockSpec((1, _T, 5), lambda b: (b, 0, 0)),
        ],
        out_specs=[
            pl.BlockSpec((1, 8, 128), lambda b: (b, 0, 0)),
            pl.BlockSpec((1, _R, _L), lambda b: (b, 0, 0)),
            pl.BlockSpec((1, _R, _L), lambda b: (b, 0, 0)),
            pl.BlockSpec((1, _R, _L), lambda b: (b, 0, 0)),
        ],
        out_shape=[
            jax.ShapeDtypeStruct((B, 8, 128), jnp.float32),
            jax.ShapeDtypeStruct((B, _R, _L), jnp.float32),
            jax.ShapeDtypeStruct((B, _R, _L), jnp.float32),
            jax.ShapeDtypeStruct((B, _R, _L), jnp.float32),
        ],
    )(loc_r, conf_r, bin_r, pri_r, targets)

    sums = jnp.sum(scal[:, 0, :4], axis=0)
    n = jnp.maximum(sums[3], 1.0)
    pos = posf.reshape(B, P) > 0.5
    neg_binary = negbf.reshape(B, P) > 0.5
    neg_multi = negmf.reshape(B, P) > 0.5
    return (sums[0] / n, sums[2] / n, sums[1] / n, pos, neg_binary, neg_multi)
